# baseline (speedup 1.0000x reference)
import functools
import jax
import jax.numpy as jnp
from jax import lax
from jax.experimental import pallas as pl
from jax.experimental.pallas import tpu as pltpu
from jax.experimental.pallas import tpu_sc as plsc

_N = 10000
_C = 128
_K = 16
_E = _N * _K
_NTILE = 16
_CH = 128
_NCH = _E // _CH
_NJ = 80
_ZROW = 4 * _N
_BLK = 1000


def _prologue_body(x_ref, pos_ref, ws_ref, wd_ref, wl_ref, wp_ref, b_ref,
                   allpg_ref, ed_ref, t_ref):
    x = x_ref[...]
    p = pos_ref[...]
    for d in range(2):
        q = jnp.dot(p, wp_ref[d], preferred_element_type=jnp.float32)
        b = b_ref[d, 0:1, :]
        t = q + b
        s = jnp.dot(x, ws_ref[d], preferred_element_type=jnp.float32) + q
        dp = jnp.dot(x, wd_ref[d], preferred_element_type=jnp.float32) + t
        v = jnp.dot(x, wl_ref[d], preferred_element_type=jnp.float32) - q
        pe = jnp.exp(-s)
        g = pe * v
        ed_ref[d] = jnp.exp(dp)
        t_ref[d] = t
        for c in range(2):
            allpg_ref[2 * d + c] = jnp.concatenate(
                [pe[:, 64 * c:64 * (c + 1)], g[:, 64 * c:64 * (c + 1)]], axis=1)


def _epilogue_body(acc_ref, ed_ref, t_ref, x_ref, out_ref):
    h = None
    for d in range(2):
        a = jnp.concatenate([acc_ref[2 * d][:, :64], acc_ref[2 * d + 1][:, :64]],
                            axis=1)
        bt = jnp.concatenate([acc_ref[2 * d][:, 64:], acc_ref[2 * d + 1][:, 64:]],
                             axis=1)
        ed = ed_ref[d]
        t = t_ref[d]
        hd = ed * (bt + t * a) / (ed * a + 1e-16)
        h = hd if h is None else jnp.maximum(h, hd)
    out_ref[...] = h + x_ref[...]


_ZCH = 80
_NZC = _N // _ZCH
_NZI = -(-_NZC // _NTILE)


def _sc_body(allpg, gidx, didx, zeros_hbm, out,
             gix_v, dix_v, uidx_v, rows_v, accum, semg0, semg1, sems0, sems1):
    c = lax.axis_index("c")
    s = lax.axis_index("s")
    semg = (semg0, semg1)
    sems = (sems0, sems1)

    def gather(j, sl):
        pltpu.async_copy(allpg.at[gix_v.at[j]], rows_v.at[sl], semg[sl])

    def wait_gather(sl):
        pltpu.make_async_copy(allpg.at[pl.ds(0, _CH)], rows_v.at[sl],
                              semg[sl]).wait()

    def unpack_didx(row, half, sl):
        for g in range(4):
            v = dix_v[row, pl.ds(half * 64 + 16 * g, 16)]
            uidx_v[sl, pl.ds(32 * g, 16)] = v & 0xFFFF
            uidx_v[sl, pl.ds(32 * g + 16, 16)] = lax.shift_right_logical(v, 16)

    def scatter(j, sl):
        pltpu.async_copy(rows_v.at[sl], accum.at[uidx_v.at[sl]], sems[sl],
                         add=True)

    def wait_scatter(sl):
        pltpu.make_async_copy(allpg.at[pl.ds(0, _CH)], rows_v.at[sl],
                              sems[sl]).wait()

    for d in range(2):
        @pl.loop(0, _NZI)
        def _zero(i):
            t = i * _NTILE + s

            @pl.when(t < _NZC)
            def _():
                pltpu.sync_copy(zeros_hbm, accum.at[pl.ds(t * _ZCH, _ZCH)])

        gbase = ((c * 2 + d) * _NTILE + s) * _NJ
        pltpu.sync_copy(gidx.at[pl.ds(gbase, _NJ)], gix_v)
        dbase = (d * _NTILE + s) * (_NJ // 2)
        pltpu.sync_copy(didx.at[pl.ds(dbase, _NJ // 2)], dix_v)
        plsc.subcore_barrier()

        unpack_didx(0, 0, 0)
        unpack_didx(0, 1, 1)

        gather(0, 0)
        gather(1, 1)

        @pl.loop(0, _NJ // 2 - 1)
        def _pair(i):
            a = 2 * i
            wait_gather(0)
            scatter(a, 0)
            wait_gather(1)
            scatter(a + 1, 1)
            wait_scatter(0)
            unpack_didx(i + 1, 0, 0)
            gather(a + 2, 0)
            wait_scatter(1)
            unpack_didx(i + 1, 1, 1)
            gather(a + 3, 1)

        a = _NJ - 2
        wait_gather(0)
        scatter(a, 0)
        wait_gather(1)
        scatter(a + 1, 1)
        wait_scatter(0)
        wait_scatter(1)
        plsc.subcore_barrier()

        @pl.loop(0, _NZI)
        def _dump(i):
            t = i * _NTILE + s

            @pl.when(t < _NZC)
            def _():
                pltpu.sync_copy(accum.at[pl.ds(t * _ZCH, _ZCH)],
                                out.at[pl.ds((2 * d + c) * _N + t * _ZCH, _ZCH)])

        plsc.subcore_barrier()


def kernel(x, pos, edge_index, W_lin_0, W_src_0, W_dst_0, W_pos_0, b_pos_0,
           W_lin_1, W_src_1, W_dst_1, W_pos_1, b_pos_1):
    f32 = jnp.float32
    ws = jnp.stack([W_src_0, W_src_1])
    wd = jnp.stack([W_dst_0, W_dst_1])
    wl = jnp.stack([W_lin_0, W_lin_1])
    wp = jnp.concatenate(
        [jnp.stack([W_pos_0, W_pos_1]), jnp.zeros((2, 5, _C), f32)], axis=1)
    bp = jnp.concatenate(
        [jnp.stack([b_pos_0, b_pos_1])[:, None, :], jnp.zeros((2, 7, _C), f32)],
        axis=1)
    posp = jnp.concatenate([pos, jnp.zeros((_N, 5), f32)], axis=1)

    ei = edge_index.reshape(2, _N, 2 * _K)
    src0 = ei[0, :, :_K].reshape(-1)
    dst0 = ei[1, :, :_K].reshape(-1)
    src1 = ei[0, :, ::2].reshape(-1)
    dst1 = ei[1, :, ::2].reshape(-1)

    def _regroup(vals, pad_val):
        a = vals.reshape(_NCH, _CH)
        a = jnp.concatenate(
            [a, jnp.full((_NJ * _NTILE - _NCH, _CH), pad_val, jnp.int32)])
        return a.reshape(_NJ, _NTILE, _CH).transpose(1, 0, 2)

    gidx = jnp.stack([
        jnp.stack([_regroup(src0, _ZROW), _regroup(src1 + 2 * _N, _ZROW)]),
        jnp.stack([_regroup(src0 + _N, _ZROW), _regroup(src1 + 3 * _N, _ZROW)]),
    ]).reshape(2 * 2 * _NTILE * _NJ, _CH)
    def _pack16(arr):
        r = arr.reshape(_NTILE, _NJ // 2, 2, 4, 2, 16)
        p = r[:, :, :, :, 0, :] | (r[:, :, :, :, 1, :] << 16)
        return p.reshape(_NTILE, _NJ // 2, _CH)

    didx = jnp.stack([
        _pack16(_regroup(dst0, 0)), _pack16(_regroup(dst1, 0))
    ]).reshape(2 * _NTILE * (_NJ // 2), _CH)
    zeros = jnp.zeros((_ZCH, _C), f32)

    nblk = _N // _BLK
    allpg, ed, t = pl.pallas_call(
        _prologue_body,
        grid=(nblk,),
        in_specs=[
            pl.BlockSpec((_BLK, _C), lambda i: (i, 0)),
            pl.BlockSpec((_BLK, 8), lambda i: (i, 0)),
            pl.BlockSpec((2, _C, _C), lambda i: (0, 0, 0)),
            pl.BlockSpec((2, _C, _C), lambda i: (0, 0, 0)),
            pl.BlockSpec((2, _C, _C), lambda i: (0, 0, 0)),
            pl.BlockSpec((2, 8, _C), lambda i: (0, 0, 0)),
            pl.BlockSpec((2, 8, _C), lambda i: (0, 0, 0)),
        ],
        out_specs=[
            pl.BlockSpec((4, _BLK, _C), lambda i: (0, i, 0)),
            pl.BlockSpec((2, _BLK, _C), lambda i: (0, i, 0)),
            pl.BlockSpec((2, _BLK, _C), lambda i: (0, i, 0)),
        ],
        out_shape=[
            jax.ShapeDtypeStruct((4, _N, _C), f32),
            jax.ShapeDtypeStruct((2, _N, _C), f32),
            jax.ShapeDtypeStruct((2, _N, _C), f32),
        ],
    )(x, posp, ws, wd, wl, wp, bp)

    sc = pl.kernel(
        _sc_body,
        out_type=jax.ShapeDtypeStruct((4 * _N, _C), f32),
        mesh=plsc.VectorSubcoreMesh(core_axis_name="c", subcore_axis_name="s"),
        scratch_types=[
            pltpu.VMEM((_NJ, _CH), jnp.int32),
            pltpu.VMEM((_NJ // 2, _CH), jnp.int32),
            pltpu.VMEM((2, _CH), jnp.int32),
            pltpu.VMEM((2, _CH, _C), f32),
            pltpu.VMEM_SHARED((_N, _C), f32),
            pltpu.SemaphoreType.DMA,
            pltpu.SemaphoreType.DMA,
            pltpu.SemaphoreType.DMA,
            pltpu.SemaphoreType.DMA,
        ],
    )
    allpg_flat = jnp.concatenate(
        [allpg.reshape(4 * _N, _C), jnp.zeros((8, _C), f32)])
    acc = sc(allpg_flat, gidx, didx, zeros)

    out = pl.pallas_call(
        _epilogue_body,
        grid=(nblk,),
        in_specs=[
            pl.BlockSpec((4, _BLK, _C), lambda i: (0, i, 0)),
            pl.BlockSpec((2, _BLK, _C), lambda i: (0, i, 0)),
            pl.BlockSpec((2, _BLK, _C), lambda i: (0, i, 0)),
            pl.BlockSpec((_BLK, _C), lambda i: (i, 0)),
        ],
        out_specs=pl.BlockSpec((_BLK, _C), lambda i: (i, 0)),
        out_shape=jax.ShapeDtypeStruct((_N, _C), f32),
    )(acc.reshape(4, _N, _C), ed, t, x)
    return out


# speedup vs baseline: 1.0644x; 1.0644x over previous
import functools
import jax
import jax.numpy as jnp
from jax import lax
from jax.experimental import pallas as pl
from jax.experimental.pallas import tpu as pltpu
from jax.experimental.pallas import tpu_sc as plsc

_N = 10000
_C = 128
_K = 16
_E = _N * _K
_NTILE = 16
_CH = 128
_NCH = _E // _CH
_NJ = 80
_ZROW = 4 * _N
_BLK = 1000


def _prologue_body(x_ref, pos_ref, ws_ref, wd_ref, wl_ref, wp_ref, b_ref,
                   allpg_ref, ed_ref, t_ref):
    x = x_ref[...]
    p = pos_ref[...]
    for d in range(2):
        q = jnp.dot(p, wp_ref[d], preferred_element_type=jnp.float32)
        b = b_ref[d, 0:1, :]
        t = q + b
        s = jnp.dot(x, ws_ref[d], preferred_element_type=jnp.float32) + q
        dp = jnp.dot(x, wd_ref[d], preferred_element_type=jnp.float32) + t
        v = jnp.dot(x, wl_ref[d], preferred_element_type=jnp.float32) - q
        pe = jnp.exp(-s)
        g = pe * v
        ed_ref[d] = jnp.exp(dp)
        t_ref[d] = t
        for c in range(2):
            allpg_ref[2 * d + c] = jnp.concatenate(
                [pe[:, 64 * c:64 * (c + 1)], g[:, 64 * c:64 * (c + 1)]], axis=1)


def _epilogue_body(acc_ref, ed_ref, t_ref, x_ref, out_ref):
    h = None
    for d in range(2):
        a = jnp.concatenate([acc_ref[2 * d][:, :64], acc_ref[2 * d + 1][:, :64]],
                            axis=1)
        bt = jnp.concatenate([acc_ref[2 * d][:, 64:], acc_ref[2 * d + 1][:, 64:]],
                             axis=1)
        ed = ed_ref[d]
        t = t_ref[d]
        hd = ed * (bt + t * a) / (ed * a + 1e-16)
        h = hd if h is None else jnp.maximum(h, hd)
    out_ref[...] = h + x_ref[...]


_ZCH = 80
_NZC = _N // _ZCH
_NZI = -(-_NZC // _NTILE)


def _sc_body(allpg, gidx, didx, zeros_hbm, out,
             gix_v, dix_v, uidx_v, rows_v, accum, semg0, semg1, sems0, sems1):
    c = lax.axis_index("c")
    s = lax.axis_index("s")
    semg = (semg0, semg1)
    sems = (sems0, sems1)

    def gather(j, sl):
        pltpu.async_copy(allpg.at[gix_v.at[j]], rows_v.at[sl], semg[sl])

    def wait_gather(sl):
        pltpu.make_async_copy(allpg.at[pl.ds(0, _CH)], rows_v.at[sl],
                              semg[sl]).wait()

    def unpack_didx(row, half, sl):
        for g in range(4):
            v = dix_v[row, pl.ds(half * 64 + 16 * g, 16)]
            uidx_v[sl, pl.ds(32 * g, 16)] = v & 0xFFFF
            uidx_v[sl, pl.ds(32 * g + 16, 16)] = lax.shift_right_logical(v, 16)

    def scatter(j, sl):
        pltpu.sync_copy(rows_v.at[sl], accum.at[uidx_v.at[sl]], add=True)

    def wait_scatter(sl):
        pass

    for d in range(2):
        @pl.loop(0, _NZI)
        def _zero(i):
            t = i * _NTILE + s

            @pl.when(t < _NZC)
            def _():
                pltpu.sync_copy(zeros_hbm, accum.at[pl.ds(t * _ZCH, _ZCH)])

        gbase = ((c * 2 + d) * _NTILE + s) * _NJ
        pltpu.sync_copy(gidx.at[pl.ds(gbase, _NJ)], gix_v)
        dbase = (d * _NTILE + s) * (_NJ // 2)
        pltpu.sync_copy(didx.at[pl.ds(dbase, _NJ // 2)], dix_v)
        plsc.subcore_barrier()

        unpack_didx(0, 0, 0)
        unpack_didx(0, 1, 1)

        gather(0, 0)
        gather(1, 1)

        @pl.loop(0, _NJ // 2 - 1)
        def _pair(i):
            a = 2 * i
            wait_gather(0)
            scatter(a, 0)
            unpack_didx(i + 1, 0, 0)
            gather(a + 2, 0)
            wait_gather(1)
            scatter(a + 1, 1)
            unpack_didx(i + 1, 1, 1)
            gather(a + 3, 1)

        a = _NJ - 2
        wait_gather(0)
        scatter(a, 0)
        wait_gather(1)
        scatter(a + 1, 1)
        wait_scatter(0)
        wait_scatter(1)
        plsc.subcore_barrier()

        @pl.loop(0, _NZI)
        def _dump(i):
            t = i * _NTILE + s

            @pl.when(t < _NZC)
            def _():
                pltpu.sync_copy(accum.at[pl.ds(t * _ZCH, _ZCH)],
                                out.at[pl.ds((2 * d + c) * _N + t * _ZCH, _ZCH)])

        plsc.subcore_barrier()


def kernel(x, pos, edge_index, W_lin_0, W_src_0, W_dst_0, W_pos_0, b_pos_0,
           W_lin_1, W_src_1, W_dst_1, W_pos_1, b_pos_1):
    f32 = jnp.float32
    ws = jnp.stack([W_src_0, W_src_1])
    wd = jnp.stack([W_dst_0, W_dst_1])
    wl = jnp.stack([W_lin_0, W_lin_1])
    wp = jnp.concatenate(
        [jnp.stack([W_pos_0, W_pos_1]), jnp.zeros((2, 5, _C), f32)], axis=1)
    bp = jnp.concatenate(
        [jnp.stack([b_pos_0, b_pos_1])[:, None, :], jnp.zeros((2, 7, _C), f32)],
        axis=1)
    posp = jnp.concatenate([pos, jnp.zeros((_N, 5), f32)], axis=1)

    ei = edge_index.reshape(2, _N, 2 * _K)
    src0 = ei[0, :, :_K].reshape(-1)
    dst0 = ei[1, :, :_K].reshape(-1)
    src1 = ei[0, :, ::2].reshape(-1)
    dst1 = ei[1, :, ::2].reshape(-1)

    def _regroup(vals, pad_val):
        a = vals.reshape(_NCH, _CH)
        a = jnp.concatenate(
            [a, jnp.full((_NJ * _NTILE - _NCH, _CH), pad_val, jnp.int32)])
        return a.reshape(_NJ, _NTILE, _CH).transpose(1, 0, 2)

    gidx = jnp.stack([
        jnp.stack([_regroup(src0, _ZROW), _regroup(src1 + 2 * _N, _ZROW)]),
        jnp.stack([_regroup(src0 + _N, _ZROW), _regroup(src1 + 3 * _N, _ZROW)]),
    ]).reshape(2 * 2 * _NTILE * _NJ, _CH)
    def _pack16(arr):
        r = arr.reshape(_NTILE, _NJ // 2, 2, 4, 2, 16)
        p = r[:, :, :, :, 0, :] | (r[:, :, :, :, 1, :] << 16)
        return p.reshape(_NTILE, _NJ // 2, _CH)

    didx = jnp.stack([
        _pack16(_regroup(dst0, 0)), _pack16(_regroup(dst1, 0))
    ]).reshape(2 * _NTILE * (_NJ // 2), _CH)
    zeros = jnp.zeros((_ZCH, _C), f32)

    nblk = _N // _BLK
    allpg, ed, t = pl.pallas_call(
        _prologue_body,
        grid=(nblk,),
        in_specs=[
            pl.BlockSpec((_BLK, _C), lambda i: (i, 0)),
            pl.BlockSpec((_BLK, 8), lambda i: (i, 0)),
            pl.BlockSpec((2, _C, _C), lambda i: (0, 0, 0)),
            pl.BlockSpec((2, _C, _C), lambda i: (0, 0, 0)),
            pl.BlockSpec((2, _C, _C), lambda i: (0, 0, 0)),
            pl.BlockSpec((2, 8, _C), lambda i: (0, 0, 0)),
            pl.BlockSpec((2, 8, _C), lambda i: (0, 0, 0)),
        ],
        out_specs=[
            pl.BlockSpec((4, _BLK, _C), lambda i: (0, i, 0)),
            pl.BlockSpec((2, _BLK, _C), lambda i: (0, i, 0)),
            pl.BlockSpec((2, _BLK, _C), lambda i: (0, i, 0)),
        ],
        out_shape=[
            jax.ShapeDtypeStruct((4, _N, _C), f32),
            jax.ShapeDtypeStruct((2, _N, _C), f32),
            jax.ShapeDtypeStruct((2, _N, _C), f32),
        ],
    )(x, posp, ws, wd, wl, wp, bp)

    sc = pl.kernel(
        _sc_body,
        out_type=jax.ShapeDtypeStruct((4 * _N, _C), f32),
        mesh=plsc.VectorSubcoreMesh(core_axis_name="c", subcore_axis_name="s"),
        scratch_types=[
            pltpu.VMEM((_NJ, _CH), jnp.int32),
            pltpu.VMEM((_NJ // 2, _CH), jnp.int32),
            pltpu.VMEM((2, _CH), jnp.int32),
            pltpu.VMEM((2, _CH, _C), f32),
            pltpu.VMEM_SHARED((_N, _C), f32),
            pltpu.SemaphoreType.DMA,
            pltpu.SemaphoreType.DMA,
            pltpu.SemaphoreType.DMA,
            pltpu.SemaphoreType.DMA,
        ],
    )
    allpg_flat = jnp.concatenate(
        [allpg.reshape(4 * _N, _C), jnp.zeros((8, _C), f32)])
    acc = sc(allpg_flat, gidx, didx, zeros)

    out = pl.pallas_call(
        _epilogue_body,
        grid=(nblk,),
        in_specs=[
            pl.BlockSpec((4, _BLK, _C), lambda i: (0, i, 0)),
            pl.BlockSpec((2, _BLK, _C), lambda i: (0, i, 0)),
            pl.BlockSpec((2, _BLK, _C), lambda i: (0, i, 0)),
            pl.BlockSpec((_BLK, _C), lambda i: (i, 0)),
        ],
        out_specs=pl.BlockSpec((_BLK, _C), lambda i: (i, 0)),
        out_shape=jax.ShapeDtypeStruct((_N, _C), f32),
    )(acc.reshape(4, _N, _C), ed, t, x)
    return out


# speedup vs baseline: 2.3035x; 2.1641x over previous
import functools
import jax
import jax.numpy as jnp
from jax import lax
from jax.experimental import pallas as pl
from jax.experimental.pallas import tpu as pltpu
from jax.experimental.pallas import tpu_sc as plsc

_N = 10000
_C = 128
_K = 16
_E = _N * _K
_NTILE = 16
_EPT = _E // _NTILE
_CH = 128
_NF = _EPT // _CH
_CT = _EPT - _NF * _CH
_BLK = 1000


def _prologue_body(x_ref, pos_ref, ws_ref, wd_ref, wl_ref, wp_ref, b_ref,
                   allpg_ref, ed_ref, t_ref):
    x = x_ref[...]
    p = pos_ref[...]
    for d in range(2):
        q = jnp.dot(p, wp_ref[d], preferred_element_type=jnp.float32)
        b = b_ref[d, 0:1, :]
        t = q + b
        s = jnp.dot(x, ws_ref[d], preferred_element_type=jnp.float32) + q
        dp = jnp.dot(x, wd_ref[d], preferred_element_type=jnp.float32) + t
        v = jnp.dot(x, wl_ref[d], preferred_element_type=jnp.float32) - q
        pe = jnp.exp(-s)
        g = pe * v
        ed_ref[d] = jnp.exp(dp)
        t_ref[d] = t
        for c in range(2):
            allpg_ref[2 * d + c] = jnp.concatenate(
                [pe[:, 64 * c:64 * (c + 1)], g[:, 64 * c:64 * (c + 1)]], axis=1)


def _epilogue_body(acc_ref, ed_ref, t_ref, x_ref, out_ref):
    h = None
    for d in range(2):
        a = jnp.concatenate([acc_ref[2 * d][:, :64], acc_ref[2 * d + 1][:, :64]],
                            axis=1)
        bt = jnp.concatenate([acc_ref[2 * d][:, 64:], acc_ref[2 * d + 1][:, 64:]],
                             axis=1)
        ed = ed_ref[d]
        t = t_ref[d]
        hd = ed * (bt + t * a) / (ed * a + 1e-16)
        h = hd if h is None else jnp.maximum(h, hd)
    out_ref[...] = h + x_ref[...]


_ZCH = 80
_NZC = _N // _ZCH
_NZI = -(-_NZC // _NTILE)


def _sc_body(allpg, srcidx, dstidx, zeros_hbm, out,
             sidx0, sidx1, didx0, didx1, rows0, rows1, sidxt, didxt, rowst,
             accum, semg0, semg1, semgt):
    c = lax.axis_index("c")
    s = lax.axis_index("s")
    sidx = (sidx0, sidx1)
    didx = (didx0, didx1)
    rows = (rows0, rows1)
    semg = (semg0, semg1)

    for d in range(2):
        @pl.loop(0, _NZI)
        def _zero(i):
            t = i * _NTILE + s

            @pl.when(t < _NZC)
            def _():
                pltpu.sync_copy(zeros_hbm, accum.at[pl.ds(t * _ZCH, _ZCH)])

        plsc.subcore_barrier()

        ebase = d * _E + s * _EPT

        def load_issue(t, sl):
            base = ebase + t * _CH
            pltpu.sync_copy(srcidx.at[pl.ds(c * 2 * _E + base, _CH)], sidx[sl])
            pltpu.sync_copy(dstidx.at[pl.ds(base, _CH)], didx[sl])
            pltpu.async_copy(allpg.at[sidx[sl]], rows[sl], semg[sl])

        def wait_scatter(sl):
            pltpu.make_async_copy(allpg.at[pl.ds(0, _CH)], rows[sl],
                                  semg[sl]).wait()
            pltpu.sync_copy(rows[sl], accum.at[didx[sl]], add=True)

        load_issue(0, 0)

        @pl.loop(0, _NF // 2 - 1)
        def _pair(i):
            t = 2 * i
            load_issue(t + 1, 1)
            wait_scatter(0)
            load_issue(t + 2, 0)
            wait_scatter(1)

        load_issue(_NF - 1, 1)
        wait_scatter(0)
        tbase = ebase + _NF * _CH
        pltpu.sync_copy(srcidx.at[pl.ds(c * 2 * _E + tbase, _CT)], sidxt)
        pltpu.sync_copy(dstidx.at[pl.ds(tbase, _CT)], didxt)
        pltpu.async_copy(allpg.at[sidxt], rowst, semgt)
        wait_scatter(1)
        pltpu.make_async_copy(allpg.at[pl.ds(0, _CT)], rowst, semgt).wait()
        pltpu.sync_copy(rowst, accum.at[didxt], add=True)

        plsc.subcore_barrier()

        @pl.loop(0, _NZI)
        def _dump(i):
            t = i * _NTILE + s

            @pl.when(t < _NZC)
            def _():
                pltpu.sync_copy(accum.at[pl.ds(t * _ZCH, _ZCH)],
                                out.at[pl.ds((2 * d + c) * _N + t * _ZCH, _ZCH)])

        plsc.subcore_barrier()


def kernel(x, pos, edge_index, W_lin_0, W_src_0, W_dst_0, W_pos_0, b_pos_0,
           W_lin_1, W_src_1, W_dst_1, W_pos_1, b_pos_1):
    f32 = jnp.float32
    ws = jnp.stack([W_src_0, W_src_1])
    wd = jnp.stack([W_dst_0, W_dst_1])
    wl = jnp.stack([W_lin_0, W_lin_1])
    wp = jnp.concatenate(
        [jnp.stack([W_pos_0, W_pos_1]), jnp.zeros((2, 5, _C), f32)], axis=1)
    bp = jnp.concatenate(
        [jnp.stack([b_pos_0, b_pos_1])[:, None, :], jnp.zeros((2, 7, _C), f32)],
        axis=1)
    posp = jnp.concatenate([pos, jnp.zeros((_N, 5), f32)], axis=1)

    ei = edge_index.reshape(2, _N, 2 * _K)
    src0 = ei[0, :, :_K].reshape(-1)
    dst0 = ei[1, :, :_K].reshape(-1)
    src1 = ei[0, :, ::2].reshape(-1)
    dst1 = ei[1, :, ::2].reshape(-1)

    srcidx = jnp.concatenate([src0, src1 + 2 * _N, src0 + _N, src1 + 3 * _N])
    dstidx = jnp.concatenate([dst0, dst1])
    zeros = jnp.zeros((_ZCH, _C), f32)

    nblk = _N // _BLK
    allpg, ed, t = pl.pallas_call(
        _prologue_body,
        grid=(nblk,),
        in_specs=[
            pl.BlockSpec((_BLK, _C), lambda i: (i, 0)),
            pl.BlockSpec((_BLK, 8), lambda i: (i, 0)),
            pl.BlockSpec((2, _C, _C), lambda i: (0, 0, 0)),
            pl.BlockSpec((2, _C, _C), lambda i: (0, 0, 0)),
            pl.BlockSpec((2, _C, _C), lambda i: (0, 0, 0)),
            pl.BlockSpec((2, 8, _C), lambda i: (0, 0, 0)),
            pl.BlockSpec((2, 8, _C), lambda i: (0, 0, 0)),
        ],
        out_specs=[
            pl.BlockSpec((4, _BLK, _C), lambda i: (0, i, 0)),
            pl.BlockSpec((2, _BLK, _C), lambda i: (0, i, 0)),
            pl.BlockSpec((2, _BLK, _C), lambda i: (0, i, 0)),
        ],
        out_shape=[
            jax.ShapeDtypeStruct((4, _N, _C), f32),
            jax.ShapeDtypeStruct((2, _N, _C), f32),
            jax.ShapeDtypeStruct((2, _N, _C), f32),
        ],
    )(x, posp, ws, wd, wl, wp, bp)

    sc = pl.kernel(
        _sc_body,
        out_type=jax.ShapeDtypeStruct((4 * _N, _C), f32),
        mesh=plsc.VectorSubcoreMesh(core_axis_name="c", subcore_axis_name="s"),
        scratch_types=[
            pltpu.VMEM((_CH,), jnp.int32),
            pltpu.VMEM((_CH,), jnp.int32),
            pltpu.VMEM((_CH,), jnp.int32),
            pltpu.VMEM((_CH,), jnp.int32),
            pltpu.VMEM((_CH, _C), f32),
            pltpu.VMEM((_CH, _C), f32),
            pltpu.VMEM((_CT,), jnp.int32),
            pltpu.VMEM((_CT,), jnp.int32),
            pltpu.VMEM((_CT, _C), f32),
            pltpu.VMEM_SHARED((_N, _C), f32),
            pltpu.SemaphoreType.DMA,
            pltpu.SemaphoreType.DMA,
            pltpu.SemaphoreType.DMA,
        ],
    )
    acc = sc(allpg.reshape(4 * _N, _C), srcidx, dstidx, zeros)

    out = pl.pallas_call(
        _epilogue_body,
        grid=(nblk,),
        in_specs=[
            pl.BlockSpec((4, _BLK, _C), lambda i: (0, i, 0)),
            pl.BlockSpec((2, _BLK, _C), lambda i: (0, i, 0)),
            pl.BlockSpec((2, _BLK, _C), lambda i: (0, i, 0)),
            pl.BlockSpec((_BLK, _C), lambda i: (i, 0)),
        ],
        out_specs=pl.BlockSpec((_BLK, _C), lambda i: (i, 0)),
        out_shape=jax.ShapeDtypeStruct((_N, _C), f32),
    )(acc.reshape(4, _N, _C), ed, t, x)
    return out


# speedup vs baseline: 2.7798x; 1.2068x over previous
import functools
import jax
import jax.numpy as jnp
from jax import lax
from jax.experimental import pallas as pl
from jax.experimental.pallas import tpu as pltpu
from jax.experimental.pallas import tpu_sc as plsc

_N = 10000
_C = 128
_K = 16
_E = _N * _K
_NTILE = 16
_EPT = _E // _NTILE
_CH = 128
_NF = _EPT // _CH
_CT = _EPT - _NF * _CH
_BLK = 1000


def _prologue_body(x_ref, pos_ref, ws_ref, wd_ref, wl_ref, wp_ref, b_ref,
                   allpg_ref, ed_ref, t_ref):
    x = x_ref[...]
    p = pos_ref[...]
    for d in range(2):
        q = jnp.dot(p, wp_ref[d], preferred_element_type=jnp.float32)
        b = b_ref[d, 0:1, :]
        t = q + b
        s = jnp.dot(x, ws_ref[d], preferred_element_type=jnp.float32) + q
        dp = jnp.dot(x, wd_ref[d], preferred_element_type=jnp.float32) + t
        v = jnp.dot(x, wl_ref[d], preferred_element_type=jnp.float32) - q
        pe = jnp.exp(-s)
        g = pe * v
        ed_ref[d] = jnp.exp(dp)
        t_ref[d] = t
        for c in range(2):
            allpg_ref[2 * d + c] = jnp.concatenate(
                [pe[:, 64 * c:64 * (c + 1)], g[:, 64 * c:64 * (c + 1)]], axis=1)


def _epilogue_body(acc_ref, ed_ref, t_ref, x_ref, out_ref):
    h = None
    for d in range(2):
        a = jnp.concatenate([acc_ref[2 * d][:, :64], acc_ref[2 * d + 1][:, :64]],
                            axis=1)
        bt = jnp.concatenate([acc_ref[2 * d][:, 64:], acc_ref[2 * d + 1][:, 64:]],
                             axis=1)
        ed = ed_ref[d]
        t = t_ref[d]
        hd = ed * (bt + t * a) / (ed * a + 1e-16)
        h = hd if h is None else jnp.maximum(h, hd)
    out_ref[...] = h + x_ref[...]


_ZCH = 80
_NZC = _N // _ZCH
_NZI = -(-_NZC // _NTILE)


def _sc_body(allpg, srcidx, dstidx, zeros_hbm, out,
             si0, si1, si2, di0, di1, di2, rows0, rows1, sidxt, didxt, rowst,
             accum, semi0, semi1, semi2, semg0, semg1, semit, semgt):
    c = lax.axis_index("c")
    s = lax.axis_index("s")
    si = (si0, si1, si2)
    di = (di0, di1, di2)
    rows = (rows0, rows1)
    semi = (semi0, semi1, semi2)
    semg = (semg0, semg1)

    for d in range(2):
        @pl.loop(0, _NZI)
        def _zero(i):
            t = i * _NTILE + s

            @pl.when(t < _NZC)
            def _():
                pltpu.sync_copy(zeros_hbm, accum.at[pl.ds(t * _ZCH, _ZCH)])

        plsc.subcore_barrier()

        ebase = d * _E + s * _EPT

        def idx_start(t, w):
            base = ebase + t * _CH
            pltpu.async_copy(srcidx.at[pl.ds(c * 2 * _E + base, _CH)], si[w],
                             semi[w])
            pltpu.async_copy(dstidx.at[pl.ds(base, _CH)], di[w], semi[w])

        def gather_start(w, r):
            pltpu.make_async_copy(srcidx.at[pl.ds(0, _CH)], si[w],
                                  semi[w]).wait()
            pltpu.make_async_copy(srcidx.at[pl.ds(0, _CH)], di[w],
                                  semi[w]).wait()
            pltpu.async_copy(allpg.at[si[w]], rows[r], semg[r])

        def scat(w, r):
            pltpu.make_async_copy(allpg.at[pl.ds(0, _CH)], rows[r],
                                  semg[r]).wait()
            pltpu.sync_copy(rows[r], accum.at[di[w]], add=True)

        def step(t, k):
            gather_start((k + 1) % 3, (k + 1) % 2)
            idx_start(t + 2, (k + 2) % 3)
            scat(k % 3, k % 2)

        idx_start(0, 0)
        gather_start(0, 0)
        idx_start(1, 1)

        @pl.loop(0, (_NF - 6) // 6)
        def _six(i):
            t0 = 6 * i
            for k in range(6):
                step(t0 + k, k)

        t0 = _NF - 6
        step(t0, 0)
        step(t0 + 1, 1)
        step(t0 + 2, 2)
        step(t0 + 3, 3)
        gather_start(2, 1)
        tbase = ebase + _NF * _CH
        pltpu.async_copy(srcidx.at[pl.ds(c * 2 * _E + tbase, _CT)], sidxt,
                         semit)
        pltpu.async_copy(dstidx.at[pl.ds(tbase, _CT)], didxt, semit)
        scat(1, 0)
        pltpu.make_async_copy(srcidx.at[pl.ds(0, _CT)], sidxt, semit).wait()
        pltpu.make_async_copy(srcidx.at[pl.ds(0, _CT)], didxt, semit).wait()
        pltpu.async_copy(allpg.at[sidxt], rowst, semgt)
        scat(2, 1)
        pltpu.make_async_copy(allpg.at[pl.ds(0, _CT)], rowst, semgt).wait()
        pltpu.sync_copy(rowst, accum.at[didxt], add=True)

        plsc.subcore_barrier()

        @pl.loop(0, _NZI)
        def _dump(i):
            t = i * _NTILE + s

            @pl.when(t < _NZC)
            def _():
                pltpu.sync_copy(accum.at[pl.ds(t * _ZCH, _ZCH)],
                                out.at[pl.ds((2 * d + c) * _N + t * _ZCH, _ZCH)])

        plsc.subcore_barrier()


def kernel(x, pos, edge_index, W_lin_0, W_src_0, W_dst_0, W_pos_0, b_pos_0,
           W_lin_1, W_src_1, W_dst_1, W_pos_1, b_pos_1):
    f32 = jnp.float32
    ws = jnp.stack([W_src_0, W_src_1])
    wd = jnp.stack([W_dst_0, W_dst_1])
    wl = jnp.stack([W_lin_0, W_lin_1])
    wp = jnp.concatenate(
        [jnp.stack([W_pos_0, W_pos_1]), jnp.zeros((2, 5, _C), f32)], axis=1)
    bp = jnp.concatenate(
        [jnp.stack([b_pos_0, b_pos_1])[:, None, :], jnp.zeros((2, 7, _C), f32)],
        axis=1)
    posp = jnp.concatenate([pos, jnp.zeros((_N, 5), f32)], axis=1)

    ei = edge_index.reshape(2, _N, 2 * _K)
    src0 = ei[0, :, :_K].reshape(-1)
    dst0 = ei[1, :, :_K].reshape(-1)
    src1 = ei[0, :, ::2].reshape(-1)
    dst1 = ei[1, :, ::2].reshape(-1)

    srcidx = jnp.concatenate([src0, src1 + 2 * _N, src0 + _N, src1 + 3 * _N])
    dstidx = jnp.concatenate([dst0, dst1])
    zeros = jnp.zeros((_ZCH, _C), f32)

    nblk = _N // _BLK
    allpg, ed, t = pl.pallas_call(
        _prologue_body,
        grid=(nblk,),
        in_specs=[
            pl.BlockSpec((_BLK, _C), lambda i: (i, 0)),
            pl.BlockSpec((_BLK, 8), lambda i: (i, 0)),
            pl.BlockSpec((2, _C, _C), lambda i: (0, 0, 0)),
            pl.BlockSpec((2, _C, _C), lambda i: (0, 0, 0)),
            pl.BlockSpec((2, _C, _C), lambda i: (0, 0, 0)),
            pl.BlockSpec((2, 8, _C), lambda i: (0, 0, 0)),
            pl.BlockSpec((2, 8, _C), lambda i: (0, 0, 0)),
        ],
        out_specs=[
            pl.BlockSpec((4, _BLK, _C), lambda i: (0, i, 0)),
            pl.BlockSpec((2, _BLK, _C), lambda i: (0, i, 0)),
            pl.BlockSpec((2, _BLK, _C), lambda i: (0, i, 0)),
        ],
        out_shape=[
            jax.ShapeDtypeStruct((4, _N, _C), f32),
            jax.ShapeDtypeStruct((2, _N, _C), f32),
            jax.ShapeDtypeStruct((2, _N, _C), f32),
        ],
    )(x, posp, ws, wd, wl, wp, bp)

    sc = pl.kernel(
        _sc_body,
        out_type=jax.ShapeDtypeStruct((4 * _N, _C), f32),
        mesh=plsc.VectorSubcoreMesh(core_axis_name="c", subcore_axis_name="s"),
        scratch_types=[
            pltpu.VMEM((_CH,), jnp.int32),
            pltpu.VMEM((_CH,), jnp.int32),
            pltpu.VMEM((_CH,), jnp.int32),
            pltpu.VMEM((_CH,), jnp.int32),
            pltpu.VMEM((_CH,), jnp.int32),
            pltpu.VMEM((_CH,), jnp.int32),
            pltpu.VMEM((_CH, _C), f32),
            pltpu.VMEM((_CH, _C), f32),
            pltpu.VMEM((_CT,), jnp.int32),
            pltpu.VMEM((_CT,), jnp.int32),
            pltpu.VMEM((_CT, _C), f32),
            pltpu.VMEM_SHARED((_N, _C), f32),
            pltpu.SemaphoreType.DMA,
            pltpu.SemaphoreType.DMA,
            pltpu.SemaphoreType.DMA,
            pltpu.SemaphoreType.DMA,
            pltpu.SemaphoreType.DMA,
            pltpu.SemaphoreType.DMA,
            pltpu.SemaphoreType.DMA,
        ],
    )
    acc = sc(allpg.reshape(4 * _N, _C), srcidx, dstidx, zeros)

    out = pl.pallas_call(
        _epilogue_body,
        grid=(nblk,),
        in_specs=[
            pl.BlockSpec((4, _BLK, _C), lambda i: (0, i, 0)),
            pl.BlockSpec((2, _BLK, _C), lambda i: (0, i, 0)),
            pl.BlockSpec((2, _BLK, _C), lambda i: (0, i, 0)),
            pl.BlockSpec((_BLK, _C), lambda i: (i, 0)),
        ],
        out_specs=pl.BlockSpec((_BLK, _C), lambda i: (i, 0)),
        out_shape=jax.ShapeDtypeStruct((_N, _C), f32),
    )(acc.reshape(4, _N, _C), ed, t, x)
    return out


# speedup vs baseline: 2.7941x; 1.0052x over previous
import functools
import jax
import jax.numpy as jnp
from jax import lax
from jax.experimental import pallas as pl
from jax.experimental.pallas import tpu as pltpu
from jax.experimental.pallas import tpu_sc as plsc

_N = 10000
_C = 128
_K = 16
_E = _N * _K
_NTILE = 16
_EPT = _E // _NTILE
_CH = 128
_NF = _EPT // _CH
_CT = _EPT - _NF * _CH
_BLK = 1000


def _prologue_body(x_ref, pos_ref, ws_ref, wd_ref, wl_ref, wp_ref, b_ref,
                   allpg_ref, ed_ref, t_ref):
    x = x_ref[...]
    p = pos_ref[...]
    for d in range(2):
        q = jnp.dot(p, wp_ref[d], preferred_element_type=jnp.float32)
        b = b_ref[d, 0:1, :]
        t = q + b
        s = jnp.dot(x, ws_ref[d], preferred_element_type=jnp.float32) + q
        dp = jnp.dot(x, wd_ref[d], preferred_element_type=jnp.float32) + t
        v = jnp.dot(x, wl_ref[d], preferred_element_type=jnp.float32) - q
        pe = jnp.exp(-s)
        g = pe * v
        ed_ref[d] = jnp.exp(dp)
        t_ref[d] = t
        for c in range(2):
            allpg_ref[2 * d + c] = jnp.concatenate(
                [pe[:, 64 * c:64 * (c + 1)], g[:, 64 * c:64 * (c + 1)]], axis=1)


def _epilogue_body(acc_ref, ed_ref, t_ref, x_ref, out_ref):
    h = None
    for d in range(2):
        a = jnp.concatenate([acc_ref[2 * d][:, :64], acc_ref[2 * d + 1][:, :64]],
                            axis=1)
        bt = jnp.concatenate([acc_ref[2 * d][:, 64:], acc_ref[2 * d + 1][:, 64:]],
                             axis=1)
        ed = ed_ref[d]
        t = t_ref[d]
        hd = ed * (bt + t * a) / (ed * a + 1e-16)
        h = hd if h is None else jnp.maximum(h, hd)
    out_ref[...] = h + x_ref[...]


_ZCH = 80
_NZC = _N // _ZCH
_NZI = -(-_NZC // _NTILE)


def _sc_body(allpg, srcidx, dstidx, zeros_hbm, out,
             si0, si1, si2, di0, di1, di2, rows0, rows1, sidxt, didxt, rowst,
             accum, semi0, semi1, semi2, semg0, semg1, semit, semgt, semz):
    c = lax.axis_index("c")
    s = lax.axis_index("s")
    si = (si0, si1, si2)
    di = (di0, di1, di2)
    rows = (rows0, rows1)
    semi = (semi0, semi1, semi2)
    semg = (semg0, semg1)

    for d in range(2):
        @pl.loop(0, _NZI)
        def _zero(i):
            t = i * _NTILE + s

            @pl.when(t < _NZC)
            def _():
                pltpu.async_copy(zeros_hbm, accum.at[pl.ds(t * _ZCH, _ZCH)],
                                 semz)

        ebase = d * _E + s * _EPT

        def idx_start(t, w):
            base = ebase + t * _CH
            pltpu.async_copy(srcidx.at[pl.ds(c * 2 * _E + base, _CH)], si[w],
                             semi[w])
            pltpu.async_copy(dstidx.at[pl.ds(base, _CH)], di[w], semi[w])

        def gather_start(w, r):
            pltpu.make_async_copy(srcidx.at[pl.ds(0, _CH)], si[w],
                                  semi[w]).wait()
            pltpu.make_async_copy(srcidx.at[pl.ds(0, _CH)], di[w],
                                  semi[w]).wait()
            pltpu.async_copy(allpg.at[si[w]], rows[r], semg[r])

        def scat(w, r):
            pltpu.make_async_copy(allpg.at[pl.ds(0, _CH)], rows[r],
                                  semg[r]).wait()
            pltpu.sync_copy(rows[r], accum.at[di[w]], add=True)

        def step(t, k):
            gather_start((k + 1) % 3, (k + 1) % 2)
            idx_start(t + 2, (k + 2) % 3)
            scat(k % 3, k % 2)

        idx_start(0, 0)
        gather_start(0, 0)
        idx_start(1, 1)

        @pl.loop(0, _NZI)
        def _zdrain(i):
            t = i * _NTILE + s

            @pl.when(t < _NZC)
            def _():
                pltpu.make_async_copy(
                    zeros_hbm, accum.at[pl.ds(t * _ZCH, _ZCH)], semz).wait()

        plsc.subcore_barrier()

        @pl.loop(0, (_NF - 6) // 6)
        def _six(i):
            t0 = 6 * i
            for k in range(6):
                step(t0 + k, k)

        t0 = _NF - 6
        step(t0, 0)
        step(t0 + 1, 1)
        step(t0 + 2, 2)
        step(t0 + 3, 3)
        gather_start(2, 1)
        tbase = ebase + _NF * _CH
        pltpu.async_copy(srcidx.at[pl.ds(c * 2 * _E + tbase, _CT)], sidxt,
                         semit)
        pltpu.async_copy(dstidx.at[pl.ds(tbase, _CT)], didxt, semit)
        scat(1, 0)
        pltpu.make_async_copy(srcidx.at[pl.ds(0, _CT)], sidxt, semit).wait()
        pltpu.make_async_copy(srcidx.at[pl.ds(0, _CT)], didxt, semit).wait()
        pltpu.async_copy(allpg.at[sidxt], rowst, semgt)
        scat(2, 1)
        pltpu.make_async_copy(allpg.at[pl.ds(0, _CT)], rowst, semgt).wait()
        pltpu.sync_copy(rowst, accum.at[didxt], add=True)

        plsc.subcore_barrier()

        @pl.loop(0, _NZI)
        def _dump(i):
            t = i * _NTILE + s

            @pl.when(t < _NZC)
            def _():
                pltpu.async_copy(
                    accum.at[pl.ds(t * _ZCH, _ZCH)],
                    out.at[pl.ds((2 * d + c) * _N + t * _ZCH, _ZCH)], semz)

        @pl.loop(0, _NZI)
        def _ddrain(i):
            t = i * _NTILE + s

            @pl.when(t < _NZC)
            def _():
                pltpu.make_async_copy(
                    accum.at[pl.ds(t * _ZCH, _ZCH)],
                    out.at[pl.ds((2 * d + c) * _N + t * _ZCH, _ZCH)],
                    semz).wait()

        plsc.subcore_barrier()


def kernel(x, pos, edge_index, W_lin_0, W_src_0, W_dst_0, W_pos_0, b_pos_0,
           W_lin_1, W_src_1, W_dst_1, W_pos_1, b_pos_1):
    f32 = jnp.float32
    ws = jnp.stack([W_src_0, W_src_1])
    wd = jnp.stack([W_dst_0, W_dst_1])
    wl = jnp.stack([W_lin_0, W_lin_1])
    wp = jnp.concatenate(
        [jnp.stack([W_pos_0, W_pos_1]), jnp.zeros((2, 5, _C), f32)], axis=1)
    bp = jnp.concatenate(
        [jnp.stack([b_pos_0, b_pos_1])[:, None, :], jnp.zeros((2, 7, _C), f32)],
        axis=1)
    posp = jnp.concatenate([pos, jnp.zeros((_N, 5), f32)], axis=1)

    ei = edge_index.reshape(2, _N, 2 * _K)
    src0 = ei[0, :, :_K].reshape(-1)
    dst0 = ei[1, :, :_K].reshape(-1)
    src1 = ei[0, :, ::2].reshape(-1)
    dst1 = ei[1, :, ::2].reshape(-1)

    srcidx = jnp.concatenate([src0, src1 + 2 * _N, src0 + _N, src1 + 3 * _N])
    dstidx = jnp.concatenate([dst0, dst1])
    zeros = jnp.zeros((_ZCH, _C), f32)

    nblk = _N // _BLK
    allpg, ed, t = pl.pallas_call(
        _prologue_body,
        grid=(nblk,),
        in_specs=[
            pl.BlockSpec((_BLK, _C), lambda i: (i, 0)),
            pl.BlockSpec((_BLK, 8), lambda i: (i, 0)),
            pl.BlockSpec((2, _C, _C), lambda i: (0, 0, 0)),
            pl.BlockSpec((2, _C, _C), lambda i: (0, 0, 0)),
            pl.BlockSpec((2, _C, _C), lambda i: (0, 0, 0)),
            pl.BlockSpec((2, 8, _C), lambda i: (0, 0, 0)),
            pl.BlockSpec((2, 8, _C), lambda i: (0, 0, 0)),
        ],
        out_specs=[
            pl.BlockSpec((4, _BLK, _C), lambda i: (0, i, 0)),
            pl.BlockSpec((2, _BLK, _C), lambda i: (0, i, 0)),
            pl.BlockSpec((2, _BLK, _C), lambda i: (0, i, 0)),
        ],
        out_shape=[
            jax.ShapeDtypeStruct((4, _N, _C), f32),
            jax.ShapeDtypeStruct((2, _N, _C), f32),
            jax.ShapeDtypeStruct((2, _N, _C), f32),
        ],
    )(x, posp, ws, wd, wl, wp, bp)

    sc = pl.kernel(
        _sc_body,
        out_type=jax.ShapeDtypeStruct((4 * _N, _C), f32),
        mesh=plsc.VectorSubcoreMesh(core_axis_name="c", subcore_axis_name="s"),
        scratch_types=[
            pltpu.VMEM((_CH,), jnp.int32),
            pltpu.VMEM((_CH,), jnp.int32),
            pltpu.VMEM((_CH,), jnp.int32),
            pltpu.VMEM((_CH,), jnp.int32),
            pltpu.VMEM((_CH,), jnp.int32),
            pltpu.VMEM((_CH,), jnp.int32),
            pltpu.VMEM((_CH, _C), f32),
            pltpu.VMEM((_CH, _C), f32),
            pltpu.VMEM((_CT,), jnp.int32),
            pltpu.VMEM((_CT,), jnp.int32),
            pltpu.VMEM((_CT, _C), f32),
            pltpu.VMEM_SHARED((_N, _C), f32),
            pltpu.SemaphoreType.DMA,
            pltpu.SemaphoreType.DMA,
            pltpu.SemaphoreType.DMA,
            pltpu.SemaphoreType.DMA,
            pltpu.SemaphoreType.DMA,
            pltpu.SemaphoreType.DMA,
            pltpu.SemaphoreType.DMA,
            pltpu.SemaphoreType.DMA,
        ],
    )
    acc = sc(allpg.reshape(4 * _N, _C), srcidx, dstidx, zeros)

    out = pl.pallas_call(
        _epilogue_body,
        grid=(nblk,),
        in_specs=[
            pl.BlockSpec((4, _BLK, _C), lambda i: (0, i, 0)),
            pl.BlockSpec((2, _BLK, _C), lambda i: (0, i, 0)),
            pl.BlockSpec((2, _BLK, _C), lambda i: (0, i, 0)),
            pl.BlockSpec((_BLK, _C), lambda i: (i, 0)),
        ],
        out_specs=pl.BlockSpec((_BLK, _C), lambda i: (i, 0)),
        out_shape=jax.ShapeDtypeStruct((_N, _C), f32),
    )(acc.reshape(4, _N, _C), ed, t, x)
    return out


# speedup vs baseline: 3.7731x; 1.3504x over previous
import functools
import jax
import jax.numpy as jnp
from jax import lax
from jax.experimental import pallas as pl
from jax.experimental.pallas import tpu as pltpu
from jax.experimental.pallas import tpu_sc as plsc

_N = 10000
_C = 128
_K = 16
_E = _N * _K
_NTILE = 16
_EPT = _E // _NTILE
_CH = 128
_NF = _EPT // _CH
_CT = _EPT - _NF * _CH
_BLK = 1000


def _prologue_body(x_ref, pos_ref, ws_ref, wd_ref, wl_ref, wp_ref, b_ref,
                   allpg_ref, ed_ref, t_ref):
    x = x_ref[...]
    p = pos_ref[...]
    for d in range(2):
        q = jnp.dot(p, wp_ref[d], preferred_element_type=jnp.float32)
        b = b_ref[d, 0:1, :]
        t = q + b
        s = jnp.dot(x, ws_ref[d], preferred_element_type=jnp.float32) + q
        dp = jnp.dot(x, wd_ref[d], preferred_element_type=jnp.float32) + t
        v = jnp.dot(x, wl_ref[d], preferred_element_type=jnp.float32) - q
        pe = jnp.exp(-s)
        g = pe * v
        ed_ref[d] = jnp.exp(dp)
        t_ref[d] = t
        for c in range(2):
            allpg_ref[2 * d + c] = jnp.concatenate(
                [pe[:, 64 * c:64 * (c + 1)], g[:, 64 * c:64 * (c + 1)]], axis=1)


def _epilogue_body(acc_ref, ed_ref, t_ref, x_ref, out_ref):
    h = None
    for d in range(2):
        a = jnp.concatenate([acc_ref[2 * d][:, :64], acc_ref[2 * d + 1][:, :64]],
                            axis=1)
        bt = jnp.concatenate([acc_ref[2 * d][:, 64:], acc_ref[2 * d + 1][:, 64:]],
                             axis=1)
        ed = ed_ref[d]
        t = t_ref[d]
        hd = ed * (bt + t * a) / (ed * a + 1e-16)
        h = hd if h is None else jnp.maximum(h, hd)
    out_ref[...] = h + x_ref[...]


_ZCH = 80
_NZC = _N // _ZCH
_NZI = -(-_NZC // _NTILE)


def _sc_body(allpg, eiraw, zeros_hbm, out,
             si0, si1, si2, di0, di1, di2, rs0, rs1, rs2, rd0, rd1, rd2,
             rows0, rows1, rst, rdt, sit, dit, rowst,
             accum, semi0, semi1, semi2, semg0, semg1, semit, semgt, semz):
    c = lax.axis_index("c")
    s = lax.axis_index("s")
    si = (si0, si1, si2)
    di = (di0, di1, di2)
    rawS = (rs0, rs1, rs2)
    rawD = (rd0, rd1, rd2)
    rows = (rows0, rows1)
    semi = (semi0, semi1, semi2)
    semg = (semg0, semg1)
    iot1 = lax.iota(jnp.int32, 16)
    iot2 = iot1 * 2
    idx_a = jnp.minimum(iot2, 15)
    idx_b = jnp.maximum(iot2 - 16, 0)
    msk8 = iot1 < 8

    def sel_even(ref, base):
        a = ref[pl.ds(base, 16)]
        b = ref[pl.ds(base + 16, 16)]
        ga = a.at[idx_a].get(mode="promise_in_bounds")
        gb = b.at[idx_b].get(mode="promise_in_bounds")
        return jnp.where(msk8, ga, gb)

    for d in range(2):
        @pl.loop(0, _NZI)
        def _zero(i):
            t = i * _NTILE + s

            @pl.when(t < _NZC)
            def _():
                pltpu.async_copy(zeros_hbm, accum.at[pl.ds(t * _ZCH, _ZCH)],
                                 semz)

        rbase0 = s * (32 * _N // _NTILE)
        off = (2 * d + c) * _N

        def idx_start(t, w):
            rbase = rbase0 + t * 2 * _CH
            pltpu.async_copy(eiraw.at[pl.ds(rbase, 2 * _CH)], rawS[w], semi[w])
            pltpu.async_copy(eiraw.at[pl.ds(2 * _K * _N + rbase, 2 * _CH)],
                             rawD[w], semi[w])

        def gather_start(w, r):
            pltpu.make_async_copy(eiraw.at[pl.ds(0, 2 * _CH)], rawS[w],
                                  semi[w]).wait()
            pltpu.make_async_copy(eiraw.at[pl.ds(0, 2 * _CH)], rawD[w],
                                  semi[w]).wait()
            for j in range(8):
                if d == 0:
                    sv = rawS[w][pl.ds(32 * j, 16)]
                    dv = rawD[w][pl.ds(32 * j, 16)]
                else:
                    sv = sel_even(rawS[w], 32 * j)
                    dv = sel_even(rawD[w], 32 * j)
                si[w][pl.ds(16 * j, 16)] = sv + off
                di[w][pl.ds(16 * j, 16)] = dv
            pltpu.async_copy(allpg.at[si[w]], rows[r], semg[r])

        def scat(w, r):
            pltpu.make_async_copy(allpg.at[pl.ds(0, _CH)], rows[r],
                                  semg[r]).wait()
            pltpu.sync_copy(rows[r], accum.at[di[w]], add=True)

        def step(t, k):
            gather_start((k + 1) % 3, (k + 1) % 2)
            idx_start(t + 2, (k + 2) % 3)
            scat(k % 3, k % 2)

        idx_start(0, 0)
        gather_start(0, 0)
        idx_start(1, 1)

        @pl.loop(0, _NZI)
        def _zdrain(i):
            t = i * _NTILE + s

            @pl.when(t < _NZC)
            def _():
                pltpu.make_async_copy(
                    zeros_hbm, accum.at[pl.ds(t * _ZCH, _ZCH)], semz).wait()

        plsc.subcore_barrier()

        @pl.loop(0, (_NF - 6) // 6)
        def _six(i):
            t0 = 6 * i
            for k in range(6):
                step(t0 + k, k)

        t0 = _NF - 6
        step(t0, 0)
        step(t0 + 1, 1)
        step(t0 + 2, 2)
        step(t0 + 3, 3)
        gather_start(2, 1)
        tbase = rbase0 + _NF * 2 * _CH
        pltpu.async_copy(eiraw.at[pl.ds(tbase, 2 * _CT)], rst, semit)
        pltpu.async_copy(eiraw.at[pl.ds(2 * _K * _N + tbase, 2 * _CT)], rdt,
                         semit)
        scat(1, 0)
        pltpu.make_async_copy(eiraw.at[pl.ds(0, 2 * _CT)], rst, semit).wait()
        pltpu.make_async_copy(eiraw.at[pl.ds(0, 2 * _CT)], rdt, semit).wait()
        if d == 0:
            sit[...] = rst[pl.ds(0, 16)] + off
            dit[...] = rdt[pl.ds(0, 16)]
        else:
            sit[...] = sel_even(rst, 0) + off
            dit[...] = sel_even(rdt, 0)
        pltpu.async_copy(allpg.at[sit], rowst, semgt)
        scat(2, 1)
        pltpu.make_async_copy(allpg.at[pl.ds(0, _CT)], rowst, semgt).wait()
        pltpu.sync_copy(rowst, accum.at[dit], add=True)

        plsc.subcore_barrier()

        @pl.loop(0, _NZI)
        def _dump(i):
            t = i * _NTILE + s

            @pl.when(t < _NZC)
            def _():
                pltpu.async_copy(
                    accum.at[pl.ds(t * _ZCH, _ZCH)],
                    out.at[pl.ds((2 * d + c) * _N + t * _ZCH, _ZCH)], semz)

        @pl.loop(0, _NZI)
        def _ddrain(i):
            t = i * _NTILE + s

            @pl.when(t < _NZC)
            def _():
                pltpu.make_async_copy(
                    accum.at[pl.ds(t * _ZCH, _ZCH)],
                    out.at[pl.ds((2 * d + c) * _N + t * _ZCH, _ZCH)],
                    semz).wait()

        plsc.subcore_barrier()


def kernel(x, pos, edge_index, W_lin_0, W_src_0, W_dst_0, W_pos_0, b_pos_0,
           W_lin_1, W_src_1, W_dst_1, W_pos_1, b_pos_1):
    f32 = jnp.float32
    ws = jnp.stack([W_src_0, W_src_1])
    wd = jnp.stack([W_dst_0, W_dst_1])
    wl = jnp.stack([W_lin_0, W_lin_1])
    wp = jnp.concatenate(
        [jnp.stack([W_pos_0, W_pos_1]), jnp.zeros((2, 5, _C), f32)], axis=1)
    bp = jnp.concatenate(
        [jnp.stack([b_pos_0, b_pos_1])[:, None, :], jnp.zeros((2, 7, _C), f32)],
        axis=1)
    posp = jnp.concatenate([pos, jnp.zeros((_N, 5), f32)], axis=1)

    eiraw = edge_index.reshape(-1)
    zeros = jnp.zeros((_ZCH, _C), f32)

    nblk = _N // _BLK
    allpg, ed, t = pl.pallas_call(
        _prologue_body,
        grid=(nblk,),
        in_specs=[
            pl.BlockSpec((_BLK, _C), lambda i: (i, 0)),
            pl.BlockSpec((_BLK, 8), lambda i: (i, 0)),
            pl.BlockSpec((2, _C, _C), lambda i: (0, 0, 0)),
            pl.BlockSpec((2, _C, _C), lambda i: (0, 0, 0)),
            pl.BlockSpec((2, _C, _C), lambda i: (0, 0, 0)),
            pl.BlockSpec((2, 8, _C), lambda i: (0, 0, 0)),
            pl.BlockSpec((2, 8, _C), lambda i: (0, 0, 0)),
        ],
        out_specs=[
            pl.BlockSpec((4, _BLK, _C), lambda i: (0, i, 0)),
            pl.BlockSpec((2, _BLK, _C), lambda i: (0, i, 0)),
            pl.BlockSpec((2, _BLK, _C), lambda i: (0, i, 0)),
        ],
        out_shape=[
            jax.ShapeDtypeStruct((4, _N, _C), f32),
            jax.ShapeDtypeStruct((2, _N, _C), f32),
            jax.ShapeDtypeStruct((2, _N, _C), f32),
        ],
    )(x, posp, ws, wd, wl, wp, bp)

    sc = pl.kernel(
        _sc_body,
        out_type=jax.ShapeDtypeStruct((4 * _N, _C), f32),
        mesh=plsc.VectorSubcoreMesh(core_axis_name="c", subcore_axis_name="s"),
        scratch_types=[
            pltpu.VMEM((_CH,), jnp.int32),
            pltpu.VMEM((_CH,), jnp.int32),
            pltpu.VMEM((_CH,), jnp.int32),
            pltpu.VMEM((_CH,), jnp.int32),
            pltpu.VMEM((_CH,), jnp.int32),
            pltpu.VMEM((_CH,), jnp.int32),
            pltpu.VMEM((2 * _CH,), jnp.int32),
            pltpu.VMEM((2 * _CH,), jnp.int32),
            pltpu.VMEM((2 * _CH,), jnp.int32),
            pltpu.VMEM((2 * _CH,), jnp.int32),
            pltpu.VMEM((2 * _CH,), jnp.int32),
            pltpu.VMEM((2 * _CH,), jnp.int32),
            pltpu.VMEM((_CH, _C), f32),
            pltpu.VMEM((_CH, _C), f32),
            pltpu.VMEM((2 * _CT,), jnp.int32),
            pltpu.VMEM((2 * _CT,), jnp.int32),
            pltpu.VMEM((_CT,), jnp.int32),
            pltpu.VMEM((_CT,), jnp.int32),
            pltpu.VMEM((_CT, _C), f32),
            pltpu.VMEM_SHARED((_N, _C), f32),
            pltpu.SemaphoreType.DMA,
            pltpu.SemaphoreType.DMA,
            pltpu.SemaphoreType.DMA,
            pltpu.SemaphoreType.DMA,
            pltpu.SemaphoreType.DMA,
            pltpu.SemaphoreType.DMA,
            pltpu.SemaphoreType.DMA,
            pltpu.SemaphoreType.DMA,
        ],
    )
    acc = sc(allpg.reshape(4 * _N, _C), eiraw, zeros)

    out = pl.pallas_call(
        _epilogue_body,
        grid=(nblk,),
        in_specs=[
            pl.BlockSpec((4, _BLK, _C), lambda i: (0, i, 0)),
            pl.BlockSpec((2, _BLK, _C), lambda i: (0, i, 0)),
            pl.BlockSpec((2, _BLK, _C), lambda i: (0, i, 0)),
            pl.BlockSpec((_BLK, _C), lambda i: (i, 0)),
        ],
        out_specs=pl.BlockSpec((_BLK, _C), lambda i: (i, 0)),
        out_shape=jax.ShapeDtypeStruct((_N, _C), f32),
    )(acc.reshape(4, _N, _C), ed, t, x)
    return out


# speedup vs baseline: 3.8297x; 1.0150x over previous
import jax
import jax.numpy as jnp
from jax import lax
from jax.experimental import pallas as pl
from jax.experimental.pallas import tpu as pltpu
from jax.experimental.pallas import tpu_sc as plsc

_N = 10000
_C = 128
_K = 16
_E = _N * _K
_NTILE = 16
_EPT = _E // _NTILE
_CH = 128
_NF = _EPT // _CH
_CT = _EPT - _NF * _CH
_BLK = 1000


def _prologue_body(x_ref, pos_ref, ws_ref, wd_ref, wl_ref, wp_ref, b_ref,
                   allpg_ref, ed_ref, t_ref):
    x = x_ref[...]
    p = pos_ref[...]
    for d in range(2):
        q = jnp.dot(p, wp_ref[d], preferred_element_type=jnp.float32)
        b = b_ref[d, 0:1, :]
        t = q + b
        s = jnp.dot(x, ws_ref[d], preferred_element_type=jnp.float32) + q
        dp = jnp.dot(x, wd_ref[d], preferred_element_type=jnp.float32) + t
        v = jnp.dot(x, wl_ref[d], preferred_element_type=jnp.float32) - q
        pe = jnp.exp(-s)
        g = pe * v
        ed_ref[d] = jnp.exp(dp)
        t_ref[d] = t
        for c in range(2):
            allpg_ref[2 * d + c] = jnp.concatenate(
                [pe[:, 64 * c:64 * (c + 1)], g[:, 64 * c:64 * (c + 1)]], axis=1)


def _epilogue_body(acc_ref, ed_ref, t_ref, x_ref, out_ref):
    h = None
    for d in range(2):
        a = jnp.concatenate([acc_ref[2 * d][:, :64], acc_ref[2 * d + 1][:, :64]],
                            axis=1)
        bt = jnp.concatenate([acc_ref[2 * d][:, 64:], acc_ref[2 * d + 1][:, 64:]],
                             axis=1)
        ed = ed_ref[d]
        t = t_ref[d]
        hd = ed * (bt + t * a) / (ed * a + 1e-16)
        h = hd if h is None else jnp.maximum(h, hd)
    out_ref[...] = h + x_ref[...]


_ZCH = 80
_NZC = _N // _ZCH
_NZI = -(-_NZC // _NTILE)


def _sc_body(allpg, eiraw, zeros_hbm, out,
             si0, si1, si2, di0, di1, di2, rs0, rs1, rs2, rd0, rd1, rd2,
             rows0, rows1, rst, rdt, sit, dit, rowst,
             accum, semi0, semi1, semi2, semg0, semg1, semit, semgt, semz):
    c = lax.axis_index("c")
    s = lax.axis_index("s")
    si = (si0, si1, si2)
    di = (di0, di1, di2)
    rawS = (rs0, rs1, rs2)
    rawD = (rd0, rd1, rd2)
    rows = (rows0, rows1)
    semi = (semi0, semi1, semi2)
    semg = (semg0, semg1)
    iot1 = lax.iota(jnp.int32, 16)
    iot2 = iot1 * 2
    idx_a = jnp.minimum(iot2, 15)
    idx_b = jnp.maximum(iot2 - 16, 0)
    msk8 = iot1 < 8

    def sel_even(ref, base):
        a = ref[pl.ds(base, 16)]
        b = ref[pl.ds(base + 16, 16)]
        ga = a.at[idx_a].get(mode="promise_in_bounds")
        gb = b.at[idx_b].get(mode="promise_in_bounds")
        return jnp.where(msk8, ga, gb)

    for d in range(2):
        @pl.loop(0, _NZI)
        def _zero(i):
            t = i * _NTILE + s

            @pl.when(t < _NZC)
            def _():
                pltpu.async_copy(zeros_hbm, accum.at[pl.ds(t * _ZCH, _ZCH)],
                                 semz)

        rbase0 = s * (32 * _N // _NTILE)
        off = (2 * d + c) * _N

        def idx_start(t, w):
            rbase = rbase0 + t * 2 * _CH
            pltpu.async_copy(eiraw.at[pl.ds(rbase, 2 * _CH)], rawS[w], semi[w])
            pltpu.async_copy(eiraw.at[pl.ds(2 * _K * _N + rbase, 2 * _CH)],
                             rawD[w], semi[w])

        def gather_start(w, r):
            pltpu.make_async_copy(eiraw.at[pl.ds(0, 2 * _CH)], rawS[w],
                                  semi[w]).wait()
            pltpu.make_async_copy(eiraw.at[pl.ds(0, 2 * _CH)], rawD[w],
                                  semi[w]).wait()
            for j in range(8):
                if d == 0:
                    sv = rawS[w][pl.ds(32 * j, 16)]
                    dv = rawD[w][pl.ds(32 * j, 16)]
                else:
                    sv = sel_even(rawS[w], 32 * j)
                    dv = sel_even(rawD[w], 32 * j)
                si[w][pl.ds(16 * j, 16)] = sv + off
                di[w][pl.ds(16 * j, 16)] = dv
            pltpu.async_copy(allpg.at[si[w]], rows[r], semg[r])

        def scat(w, r):
            pltpu.make_async_copy(allpg.at[pl.ds(0, _CH)], rows[r],
                                  semg[r]).wait()
            pltpu.sync_copy(rows[r], accum.at[di[w]], add=True)

        def step(t, k):
            gather_start((k + 1) % 3, (k + 1) % 2)
            idx_start(t + 2, (k + 2) % 3)
            scat(k % 3, k % 2)

        idx_start(0, 0)
        gather_start(0, 0)
        idx_start(1, 1)

        @pl.loop(0, _NZI)
        def _zdrain(i):
            t = i * _NTILE + s

            @pl.when(t < _NZC)
            def _():
                pltpu.make_async_copy(
                    zeros_hbm, accum.at[pl.ds(t * _ZCH, _ZCH)], semz).wait()

        plsc.subcore_barrier()

        @pl.loop(0, (_NF - 6) // 6)
        def _six(i):
            t0 = 6 * i
            for k in range(6):
                step(t0 + k, k)

        t0 = _NF - 6
        step(t0, 0)
        step(t0 + 1, 1)
        step(t0 + 2, 2)
        step(t0 + 3, 3)
        gather_start(2, 1)
        tbase = rbase0 + _NF * 2 * _CH
        pltpu.async_copy(eiraw.at[pl.ds(tbase, 2 * _CT)], rst, semit)
        pltpu.async_copy(eiraw.at[pl.ds(2 * _K * _N + tbase, 2 * _CT)], rdt,
                         semit)
        scat(1, 0)
        pltpu.make_async_copy(eiraw.at[pl.ds(0, 2 * _CT)], rst, semit).wait()
        pltpu.make_async_copy(eiraw.at[pl.ds(0, 2 * _CT)], rdt, semit).wait()
        if d == 0:
            sit[...] = rst[pl.ds(0, 16)] + off
            dit[...] = rdt[pl.ds(0, 16)]
        else:
            sit[...] = sel_even(rst, 0) + off
            dit[...] = sel_even(rdt, 0)
        pltpu.async_copy(allpg.at[sit], rowst, semgt)
        scat(2, 1)
        pltpu.make_async_copy(allpg.at[pl.ds(0, _CT)], rowst, semgt).wait()
        pltpu.sync_copy(rowst, accum.at[dit], add=True)

        plsc.subcore_barrier()

        @pl.loop(0, _NZI)
        def _dump(i):
            t = i * _NTILE + s

            @pl.when(t < _NZC)
            def _():
                pltpu.async_copy(
                    accum.at[pl.ds(t * _ZCH, _ZCH)],
                    out.at[pl.ds((2 * d + c) * _N + t * _ZCH, _ZCH)], semz)

        @pl.loop(0, _NZI)
        def _ddrain(i):
            t = i * _NTILE + s

            @pl.when(t < _NZC)
            def _():
                pltpu.make_async_copy(
                    accum.at[pl.ds(t * _ZCH, _ZCH)],
                    out.at[pl.ds((2 * d + c) * _N + t * _ZCH, _ZCH)],
                    semz).wait()

        plsc.subcore_barrier()


def kernel(x, pos, edge_index, W_lin_0, W_src_0, W_dst_0, W_pos_0, b_pos_0,
           W_lin_1, W_src_1, W_dst_1, W_pos_1, b_pos_1):
    f32 = jnp.float32
    ws = jnp.stack([W_src_0, W_src_1])
    wd = jnp.stack([W_dst_0, W_dst_1])
    wl = jnp.stack([W_lin_0, W_lin_1])
    wp = jnp.concatenate(
        [jnp.stack([W_pos_0, W_pos_1]), jnp.zeros((2, 5, _C), f32)], axis=1)
    bp = jnp.concatenate(
        [jnp.stack([b_pos_0, b_pos_1])[:, None, :], jnp.zeros((2, 7, _C), f32)],
        axis=1)
    posp = jnp.concatenate([pos, jnp.zeros((_N, 5), f32)], axis=1)

    eiraw = edge_index.reshape(-1)
    zeros = jnp.zeros((_ZCH, _C), f32)

    nblk = _N // _BLK
    allpg, ed, t = pl.pallas_call(
        _prologue_body,
        grid=(nblk,),
        in_specs=[
            pl.BlockSpec((_BLK, _C), lambda i: (i, 0)),
            pl.BlockSpec((_BLK, 8), lambda i: (i, 0)),
            pl.BlockSpec((2, _C, _C), lambda i: (0, 0, 0)),
            pl.BlockSpec((2, _C, _C), lambda i: (0, 0, 0)),
            pl.BlockSpec((2, _C, _C), lambda i: (0, 0, 0)),
            pl.BlockSpec((2, 8, _C), lambda i: (0, 0, 0)),
            pl.BlockSpec((2, 8, _C), lambda i: (0, 0, 0)),
        ],
        out_specs=[
            pl.BlockSpec((4, _BLK, _C), lambda i: (0, i, 0)),
            pl.BlockSpec((2, _BLK, _C), lambda i: (0, i, 0)),
            pl.BlockSpec((2, _BLK, _C), lambda i: (0, i, 0)),
        ],
        out_shape=[
            jax.ShapeDtypeStruct((4, _N, _C), f32),
            jax.ShapeDtypeStruct((2, _N, _C), f32),
            jax.ShapeDtypeStruct((2, _N, _C), f32),
        ],
    )(x, posp, ws, wd, wl, wp, bp)

    sc = pl.kernel(
        _sc_body,
        out_type=jax.ShapeDtypeStruct((4 * _N, _C), f32),
        mesh=plsc.VectorSubcoreMesh(core_axis_name="c", subcore_axis_name="s"),
        scratch_types=[
            pltpu.VMEM((_CH,), jnp.int32),
            pltpu.VMEM((_CH,), jnp.int32),
            pltpu.VMEM((_CH,), jnp.int32),
            pltpu.VMEM((_CH,), jnp.int32),
            pltpu.VMEM((_CH,), jnp.int32),
            pltpu.VMEM((_CH,), jnp.int32),
            pltpu.VMEM((2 * _CH,), jnp.int32),
            pltpu.VMEM((2 * _CH,), jnp.int32),
            pltpu.VMEM((2 * _CH,), jnp.int32),
            pltpu.VMEM((2 * _CH,), jnp.int32),
            pltpu.VMEM((2 * _CH,), jnp.int32),
            pltpu.VMEM((2 * _CH,), jnp.int32),
            pltpu.VMEM((_CH, _C), f32),
            pltpu.VMEM((_CH, _C), f32),
            pltpu.VMEM((2 * _CT,), jnp.int32),
            pltpu.VMEM((2 * _CT,), jnp.int32),
            pltpu.VMEM((_CT,), jnp.int32),
            pltpu.VMEM((_CT,), jnp.int32),
            pltpu.VMEM((_CT, _C), f32),
            pltpu.VMEM_SHARED((_N, _C), f32),
            pltpu.SemaphoreType.DMA,
            pltpu.SemaphoreType.DMA,
            pltpu.SemaphoreType.DMA,
            pltpu.SemaphoreType.DMA,
            pltpu.SemaphoreType.DMA,
            pltpu.SemaphoreType.DMA,
            pltpu.SemaphoreType.DMA,
            pltpu.SemaphoreType.DMA,
        ],
    )
    acc = sc(allpg.reshape(4 * _N, _C), eiraw, zeros)

    out = pl.pallas_call(
        _epilogue_body,
        grid=(nblk,),
        in_specs=[
            pl.BlockSpec((4, _BLK, _C), lambda i: (0, i, 0)),
            pl.BlockSpec((2, _BLK, _C), lambda i: (0, i, 0)),
            pl.BlockSpec((2, _BLK, _C), lambda i: (0, i, 0)),
            pl.BlockSpec((_BLK, _C), lambda i: (i, 0)),
        ],
        out_specs=pl.BlockSpec((_BLK, _C), lambda i: (i, 0)),
        out_shape=jax.ShapeDtypeStruct((_N, _C), f32),
    )(acc.reshape(4, _N, _C), ed, t, x)
    return out
